# Initial kernel scaffold; baseline (speedup 1.0000x reference)
import jax
import jax.numpy as jnp
from jax.experimental import pallas as pl


def kernel(x, edge_index, pos, edge_attr, W1a, R1a, B1a, W1b, R1b, B1b, W2a, R2a, B2a, W2b, R2b, B2b, W3a, R3a, B3a, W3b, R3b, B3b, Wfr, Rfr, Bfr, Wp1, Rp1, Bp1):
    raise NotImplementedError("write your pallas kernel here")


# speedup vs baseline: 1.6059x; 1.6059x over previous
import functools

import jax
import jax.numpy as jnp
from jax import lax
from jax.experimental import pallas as pl
from jax.experimental.pallas import tpu as pltpu
from jax.experimental.pallas import tpu_sc as plsc

NC = 2
NS = 16
NW = NC * NS
LANES = 16


def _round_up(v, m):
    return (v + m - 1) // m * m


def _sc_gather(table, idx, shift):
    E = idx.shape[0]
    n, D = table.shape
    per_w = E // NW
    C = 1000
    n_it = per_w // C
    n_win = -(-C // LANES)
    mesh = plsc.VectorSubcoreMesh(core_axis_name="c", subcore_axis_name="s", num_cores=NC, num_subcores=NS)

    scratch = [
        pltpu.VMEM((C,), jnp.int32),
        pltpu.VMEM((C, D), jnp.float32),
        pltpu.SemaphoreType.DMA,
    ]
    if shift:
        scratch.insert(1, pltpu.VMEM((C,), jnp.int32))

    @functools.partial(
        pl.kernel,
        mesh=mesh,
        out_type=jax.ShapeDtypeStruct((E, D), jnp.float32),
        compiler_params=pltpu.CompilerParams(use_tc_tiling_on_sc=False),
        scratch_types=scratch,
    )
    def k(tab_hbm, idx_hbm, out_hbm, *rest):
        if shift:
            idx_v, idx2_v, rows_v, sem = rest
        else:
            idx_v, rows_v, sem = rest
            idx2_v = idx_v
        wid = lax.axis_index("s") * NC + lax.axis_index("c")
        wbase = wid * per_w

        def body(i, carry):
            base = wbase + i * C
            pltpu.sync_copy(idx_hbm.at[pl.ds(base, C)], idx_v)
            if shift:
                def sh(j, c2):
                    sl = pl.ds(jnp.minimum(j * LANES, C - LANES), LANES)
                    idx2_v[sl] = lax.shift_right_logical(idx_v[sl], shift)
                    return c2
                lax.fori_loop(0, n_win, sh, 0)
            pltpu.async_copy(tab_hbm.at[idx2_v], rows_v, sem).wait()
            pltpu.sync_copy(rows_v, out_hbm.at[pl.ds(base, C)])
            return carry

        lax.fori_loop(0, n_it, body, 0)

    return k(table, idx)


def _sc_scatter(msg, src, dst, n_pad, level, with_cnt):
    E = msg.shape[0]
    D = msg.shape[1] if msg.ndim == 2 else 1
    per_w = E // NW
    C = 1000
    n_it = per_w // C
    rpw = n_pad // NS
    mesh = plsc.VectorSubcoreMesh(core_axis_name="c", subcore_axis_name="s", num_cores=NC, num_subcores=NS)

    msg_shape = (E, D) if msg.ndim == 2 else (E,)
    acc_shape = (n_pad, D) if msg.ndim == 2 else (n_pad,)
    out_shape = (NC, n_pad, D) if msg.ndim == 2 else (NC, n_pad)
    rows_shape = (C, D) if msg.ndim == 2 else (C,)

    n_win = -(-C // LANES)
    need_tr = level > 0

    out_type = [jax.ShapeDtypeStruct(out_shape, jnp.float32)]
    scratch = [
        pltpu.VMEM((C,), jnp.int32),
        pltpu.VMEM(rows_shape, jnp.float32),
        pltpu.VMEM_SHARED(acc_shape, jnp.float32),
    ]
    if need_tr:
        scratch.append(pltpu.VMEM((C,), jnp.int32))
    if with_cnt:
        out_type.append(jax.ShapeDtypeStruct((NC, n_pad), jnp.float32))
        scratch += [
            pltpu.VMEM((C,), jnp.float32),
            pltpu.VMEM_SHARED((n_pad,), jnp.float32),
        ]
        if need_tr:
            scratch += [
                pltpu.VMEM((C,), jnp.int32),
                pltpu.VMEM((C,), jnp.int32),
            ]

    zD = jnp.zeros(acc_shape, jnp.float32)
    z1 = jnp.zeros((n_pad,), jnp.float32)

    def body(*refs):
        (msg_h, src_h, dst_h, zD_h, z1_h), refs = refs[:5], refs[5:]
        aggP = refs[0]
        cntP = refs[1] if with_cnt else None
        refs = refs[2:] if with_cnt else refs[1:]
        (didx_v, rows_v, accD), refs = refs[:3], refs[3:]
        if need_tr:
            didx2_v, refs = refs[0], refs[1:]
        else:
            didx2_v = didx_v
        if with_cnt:
            (ones_v, acc1), refs = refs[:2], refs[2:]
            if need_tr:
                sidx_v, cidx_v = refs
            else:
                cidx_v = didx_v
        c = lax.axis_index("c")
        s = lax.axis_index("s")
        wid = s * NC + c
        pltpu.sync_copy(zD_h.at[pl.ds(s * rpw, rpw)], accD.at[pl.ds(s * rpw, rpw)])
        if with_cnt:
            pltpu.sync_copy(z1_h.at[pl.ds(s * rpw, rpw)], acc1.at[pl.ds(s * rpw, rpw)])

            def fill_ones(j, c2):
                sl = pl.ds(jnp.minimum(j * LANES, C - LANES), LANES)
                ones_v[sl] = jnp.full((LANES,), 1.0, jnp.float32)
                return c2
            lax.fori_loop(0, n_win, fill_ones, 0)
        plsc.subcore_barrier()

        def chunk(i, carry):
            base = wid * per_w + i * C
            pltpu.sync_copy(dst_h.at[pl.ds(base, C)], didx_v)
            if with_cnt and need_tr:
                pltpu.sync_copy(src_h.at[pl.ds(base, C)], sidx_v)

            if need_tr:
                def tr(j, c2):
                    sl = pl.ds(jnp.minimum(j * LANES, C - LANES), LANES)
                    dv = didx_v[sl]
                    d = lax.shift_right_logical(dv, level)
                    didx2_v[sl] = d
                    if with_cnt:
                        sv = sidx_v[sl]
                        if level == 1:
                            valid = lax.shift_right_logical(sv, 1) != d
                            cidx_v[sl] = jnp.where(valid, d, n_pad - 1)
                        else:
                            v1 = (lax.shift_right_logical(sv, 1)
                                  != lax.shift_right_logical(dv, 1))
                            v2 = lax.shift_right_logical(sv, 2) != d
                            cidx_v[sl] = jnp.where(v1 & v2, d, n_pad - 1)
                    return c2

                lax.fori_loop(0, n_win, tr, 0)
            pltpu.sync_copy(msg_h.at[pl.ds(base, C)], rows_v)
            pltpu.sync_copy(rows_v, accD.at[didx2_v], add=True)
            if with_cnt:
                pltpu.sync_copy(ones_v, acc1.at[cidx_v], add=True)
            return carry

        lax.fori_loop(0, n_it, chunk, 0)
        plsc.subcore_barrier()
        pltpu.sync_copy(accD.at[pl.ds(s * rpw, rpw)],
                        aggP.at[c].at[pl.ds(s * rpw, rpw)])
        if with_cnt:
            pltpu.sync_copy(acc1.at[pl.ds(s * rpw, rpw)],
                            cntP.at[c].at[pl.ds(s * rpw, rpw)])

    k = pl.kernel(body, out_type=tuple(out_type), mesh=mesh,
                  compiler_params=pltpu.CompilerParams(use_tc_tiling_on_sc=False),
                  scratch_types=scratch)
    return k(msg, src, dst, zD, z1)


def _tc_conv_edge(xs, pse, src2d, dst2d, mx, Wf, level, K, b_const):
    E, ci = xs.shape
    co = Wf.shape[1]
    Ksq = K * K
    Eb = 640
    grid = E // Eb

    def body(xs_ref, pse_ref, src_ref, dst_ref, mx_ref, w_ref, out_ref):
        a = 0.5 / mx_ref[0, 0]
        ps = pse_ref[...] * a + b_const
        u = jnp.clip(ps, 0.0, 1.0) * (K - 1)
        i0 = jnp.clip(jnp.floor(u), 0.0, float(K - 2))
        frac = u - i0
        i00 = i0[:, 0:1]
        i01 = i0[:, 1:2]
        f0 = frac[:, 0:1]
        f1 = frac[:, 1:2]
        iota_k = lax.broadcasted_iota(jnp.int32, (Eb, Ksq), 1).astype(jnp.float32)
        coeffs = jnp.zeros((Eb, Ksq), jnp.float32)
        for b0 in (0, 1):
            for b1 in (0, 1):
                cidx = (i00 + b0) * K + (i01 + b1)
                fc = (f0 if b0 else 1.0 - f0) * (f1 if b1 else 1.0 - f1)
                coeffs = coeffs + jnp.where(iota_k == cidx, fc, 0.0)
        xsv = xs_ref[...]
        pieces = [coeffs[:, kk:kk + 1] * xsv for kk in range(Ksq)]
        xk = jnp.concatenate(pieces, axis=1)
        msg = jnp.dot(xk, w_ref[...], preferred_element_type=jnp.float32)
        if level == 1:
            sv = src_ref[...]
            dv = dst_ref[...]
            val = (sv // 2 != dv // 2).astype(jnp.float32)
            msg = msg * val
        elif level == 2:
            sv = src_ref[...]
            dv = dst_ref[...]
            val = ((sv // 2 != dv // 2) & (sv // 4 != dv // 4)).astype(jnp.float32)
            msg = msg * val
        out_ref[...] = msg

    return pl.pallas_call(
        body,
        grid=(grid,),
        in_specs=[
            pl.BlockSpec((Eb, ci), lambda i: (i, 0)),
            pl.BlockSpec((Eb, 2), lambda i: (i, 0)),
            pl.BlockSpec((Eb, 1), lambda i: (i, 0)),
            pl.BlockSpec((Eb, 1), lambda i: (i, 0)),
            pl.BlockSpec((1, 1), lambda i: (0, 0), memory_space=pltpu.SMEM),
            pl.BlockSpec((Ksq * ci, co), lambda i: (0, 0)),
        ],
        out_specs=pl.BlockSpec((Eb, co), lambda i: (i, 0)),
        out_shape=jax.ShapeDtypeStruct((E, co), jnp.float32),
    )(xs, pse, src2d, dst2d, mx, Wf)


def _tc_conv_post(aggP, cntP, x, root, bias):
    n, ci = x.shape
    co = root.shape[1]

    def body(agg_ref, cnt_ref, x_ref, r_ref, b_ref, out_ref):
        agg = agg_ref[0] + agg_ref[1]
        cnt = cnt_ref[0] + cnt_ref[1]
        agg = agg / jnp.maximum(cnt, 1.0)
        if ci == 1:
            rootmm = x_ref[...] * r_ref[...]
        else:
            rootmm = jnp.dot(x_ref[...], r_ref[...],
                             preferred_element_type=jnp.float32)
        h = agg + rootmm + b_ref[...]
        out_ref[...] = jnp.where(h > 0, h, jnp.exp(h) - 1.0)

    return pl.pallas_call(
        body,
        out_shape=jax.ShapeDtypeStruct((n, co), jnp.float32),
    )(aggP, cntP, x, root, bias.reshape(1, co))


def _tc_pool(hr):
    n2, c2 = hr.shape
    co = c2 // 2

    def body(h_ref, out_ref):
        h = h_ref[...]
        out_ref[...] = jnp.maximum(h[:, :co], h[:, co:])

    return pl.pallas_call(
        body, out_shape=jax.ShapeDtypeStruct((n2, co), jnp.float32))(hr)


def _tc_pos_pool(pr):
    n2 = pr.shape[0]

    def body(p_ref, out_ref):
        p = p_ref[...]
        out_ref[...] = (p[:, :2] + p[:, 2:]) * 0.5

    return pl.pallas_call(
        body, out_shape=jax.ShapeDtypeStruct((n2, 2), jnp.float32))(pr)


def _tc_cart(ps, pd):
    E = ps.shape[0]
    Eb = 2000
    grid = E // Eb

    def body(ps_ref, pd_ref, cart_ref, mx_ref):
        cart = ps_ref[...] - pd_ref[...]
        cart_ref[...] = cart
        m = jnp.max(jnp.abs(cart))

        @pl.when(pl.program_id(0) == 0)
        def _init():
            mx_ref[0, 0] = m

        @pl.when(pl.program_id(0) > 0)
        def _acc():
            mx_ref[0, 0] = jnp.maximum(mx_ref[0, 0], m)

    return pl.pallas_call(
        body,
        grid=(grid,),
        in_specs=[
            pl.BlockSpec((Eb, 2), lambda i: (i, 0)),
            pl.BlockSpec((Eb, 2), lambda i: (i, 0)),
        ],
        out_specs=(
            pl.BlockSpec((Eb, 2), lambda i: (i, 0)),
            pl.BlockSpec((1, 1), lambda i: (0, 0), memory_space=pltpu.SMEM),
        ),
        out_shape=(
            jax.ShapeDtypeStruct((E, 2), jnp.float32),
            jax.ShapeDtypeStruct((1, 1), jnp.float32),
        ),
    )(ps, pd)


def _tc_final(h2, ps2):
    n2 = h2.shape[0]

    def body(h_ref, p_ref, out_ref):
        h = h_ref[...]
        p0 = p_ref[:, 0:1]
        p1 = p_ref[:, 1:2]
        o = jnp.concatenate([h + p0, h + p0, h + p1, h + p1], axis=1)
        out_ref[...] = 1.0 / (1.0 + jnp.exp(-o))

    return pl.pallas_call(
        body, out_shape=jax.ShapeDtypeStruct((n2, 4), jnp.float32))(h2, ps2)


def kernel(x, edge_index, pos, edge_attr,
           W1a, R1a, B1a, W1b, R1b, B1b, W2a, R2a, B2a, W2b, R2b, B2b,
           W3a, R3a, B3a, W3b, R3b, B3b, Wfr, Rfr, Bfr, Wp1, Rp1, Bp1):
    n0 = x.shape[0]
    E = edge_index.shape[1]
    src = edge_index[0].astype(jnp.int32)
    dst = edge_index[1].astype(jnp.int32)
    src2d = src.reshape(E, 1)
    dst2d = dst.reshape(E, 1)
    mx0 = jnp.full((1, 1), 0.5, jnp.float32)

    n1 = n0 // 2
    n2 = n1 // 2
    n0p = _round_up(n0, 128)
    n1p = _round_up(n1, 128)
    n2p = _round_up(n2, 128)

    xs = _sc_gather(x, src, 0)
    msg = _tc_conv_edge(xs, edge_attr, src2d, dst2d, mx0,
                        W1a.reshape(25 * 1, 32), 0, 5, 0.0)
    aggP, cnt0P = _sc_scatter(msg, src, dst, n0p, 0, True)
    cnt0 = cnt0P[:, :n0].reshape(NC, n0, 1)
    h = _tc_conv_post(aggP[:, :n0], cnt0, x, R1a, B1a)

    xs = _sc_gather(h, src, 0)
    msg = _tc_conv_edge(xs, edge_attr, src2d, dst2d, mx0,
                        W1b.reshape(25 * 32, 32), 0, 5, 0.0)
    aggP, = _sc_scatter(msg, src, dst, n0p, 0, False)
    h = _tc_conv_post(aggP[:, :n0], cnt0, h, R1b, B1b)

    x1 = _tc_pool(h.reshape(n1, 64))
    pos1 = _tc_pos_pool(pos.reshape(n1, 4))

    ps_g = _sc_gather(pos1, src, 1)
    pd_g = _sc_gather(pos1, dst, 1)
    cart1, mx1 = _tc_cart(ps_g, pd_g)

    xs = _sc_gather(x1, src, 1)
    msg = _tc_conv_edge(xs, cart1, src2d, dst2d, mx1,
                        W2a.reshape(9 * 32, 64), 1, 3, 0.5)
    aggP, cnt1P = _sc_scatter(msg, src, dst, n1p, 1, True)
    cnt1 = cnt1P[:, :n1].reshape(NC, n1, 1)
    h1 = _tc_conv_post(aggP[:, :n1], cnt1, x1, R2a, B2a)

    xs = _sc_gather(h1, src, 1)
    msg = _tc_conv_edge(xs, cart1, src2d, dst2d, mx1,
                        W2b.reshape(9 * 64, 64), 1, 3, 0.5)
    aggP, = _sc_scatter(msg, src, dst, n1p, 1, False)
    h1 = _tc_conv_post(aggP[:, :n1], cnt1, h1, R2b, B2b)

    x2 = _tc_pool(h1.reshape(n2, 128))
    pos2 = _tc_pos_pool(pos1.reshape(n2, 4))

    ps_g = _sc_gather(pos2, src, 2)
    pd_g = _sc_gather(pos2, dst, 2)
    cart2, mx2 = _tc_cart(ps_g, pd_g)

    xs = _sc_gather(x2, src, 2)
    msg = _tc_conv_edge(xs, cart2, src2d, dst2d, mx2,
                        W3a.reshape(9 * 64, 64), 2, 3, 0.5)
    aggP, cnt2P = _sc_scatter(msg, src, dst, n2p, 2, True)
    cnt2 = cnt2P[:, :n2].reshape(NC, n2, 1)
    h2 = _tc_conv_post(aggP[:, :n2], cnt2, x2, R3a, B3a)

    xs = _sc_gather(h2, src, 2)
    msg = _tc_conv_edge(xs, cart2, src2d, dst2d, mx2,
                        W3b.reshape(9 * 64, 64), 2, 3, 0.5)
    aggP, = _sc_scatter(msg, src, dst, n2p, 2, False)
    h2 = _tc_conv_post(aggP[:, :n2], cnt2, h2, R3b, B3b)

    xs = _sc_gather(h2, src, 2)
    msg = _tc_conv_edge(xs, cart2, src2d, dst2d, mx2,
                        Wfr.reshape(9 * 64, 1), 2, 3, 0.5)
    msg1 = msg.reshape(E)
    aggP, = _sc_scatter(msg1, src, dst, n2p, 2, False)
    hfr = _tc_conv_post(aggP[:, :n2].reshape(NC, n2, 1), cnt2, h2, Rfr, Bfr)

    xs = _sc_gather(x1, src, 1)
    msg = _tc_conv_edge(xs, cart1, src2d, dst2d, mx1,
                        Wp1.reshape(9 * 32, 1), 1, 3, 0.5)
    msg1 = msg.reshape(E)
    aggP, = _sc_scatter(msg1, src, dst, n1p, 1, False)
    hp = _tc_conv_post(aggP[:, :n1].reshape(NC, n1, 1), cnt1, x1, Rp1, Bp1)

    out4 = _tc_final(hfr, hp.reshape(n2, 2))
    return out4.reshape(n0, 1)


# speedup vs baseline: 1.6794x; 1.0458x over previous
import functools

import jax
import jax.numpy as jnp
from jax import lax
from jax.experimental import pallas as pl
from jax.experimental.pallas import tpu as pltpu
from jax.experimental.pallas import tpu_sc as plsc

NC = 2
NS = 16
NW = NC * NS
LANES = 16


def _round_up(v, m):
    return (v + m - 1) // m * m


def _sc_gather(table, idx, shift):
    E = idx.shape[0]
    n, D = table.shape
    per_w = E // NW
    C = 1000
    n_it = per_w // C
    n_win = -(-C // LANES)
    mesh = plsc.VectorSubcoreMesh(core_axis_name="c", subcore_axis_name="s", num_cores=NC, num_subcores=NS)

    scratch = [
        pltpu.VMEM((C,), jnp.int32),
        pltpu.VMEM((C, D), jnp.float32),
        pltpu.SemaphoreType.DMA,
    ]
    if shift:
        scratch.insert(1, pltpu.VMEM((C,), jnp.int32))

    @functools.partial(
        pl.kernel,
        mesh=mesh,
        out_type=jax.ShapeDtypeStruct((E, D), jnp.float32),
        compiler_params=pltpu.CompilerParams(use_tc_tiling_on_sc=False),
        scratch_types=scratch,
    )
    def k(tab_hbm, idx_hbm, out_hbm, *rest):
        if shift:
            idx_v, idx2_v, rows_v, sem = rest
        else:
            idx_v, rows_v, sem = rest
            idx2_v = idx_v
        wid = lax.axis_index("s") * NC + lax.axis_index("c")
        wbase = wid * per_w

        def body(i, carry):
            base = wbase + i * C
            pltpu.sync_copy(idx_hbm.at[pl.ds(base, C)], idx_v)
            if shift:
                def sh(j, c2):
                    sl = pl.ds(jnp.minimum(j * LANES, C - LANES), LANES)
                    idx2_v[sl] = lax.shift_right_logical(idx_v[sl], shift)
                    return c2
                lax.fori_loop(0, n_win, sh, 0)
            pltpu.async_copy(tab_hbm.at[idx2_v], rows_v, sem).wait()
            pltpu.sync_copy(rows_v, out_hbm.at[pl.ds(base, C)])
            return carry

        lax.fori_loop(0, n_it, body, 0)

    return k(table, idx)


def _sc_gather_pair(table, src, dst, shift):
    E = src.shape[0]
    n, D = table.shape
    per_w = E // NW
    C = 1000
    n_it = per_w // C
    n_win = -(-C // LANES)
    mesh = plsc.VectorSubcoreMesh(core_axis_name="c", subcore_axis_name="s", num_cores=NC, num_subcores=NS)

    @functools.partial(
        pl.kernel,
        mesh=mesh,
        out_type=(jax.ShapeDtypeStruct((E, D), jnp.float32),
                  jax.ShapeDtypeStruct((E, D), jnp.float32)),
        compiler_params=pltpu.CompilerParams(use_tc_tiling_on_sc=False),
        scratch_types=[
            pltpu.VMEM((C,), jnp.int32),
            pltpu.VMEM((C,), jnp.int32),
            pltpu.VMEM((C, D), jnp.float32),
            pltpu.SemaphoreType.DMA,
        ],
    )
    def k(tab_hbm, src_hbm, dst_hbm, out0_hbm, out1_hbm, idx_v, idx2_v, rows_v, sem):
        wid = lax.axis_index("s") * NC + lax.axis_index("c")
        wbase = wid * per_w

        def body(i, carry):
            base = wbase + i * C
            for idx_hbm, out_hbm in ((src_hbm, out0_hbm), (dst_hbm, out1_hbm)):
                pltpu.sync_copy(idx_hbm.at[pl.ds(base, C)], idx_v)

                def sh(j, c2):
                    sl = pl.ds(jnp.minimum(j * LANES, C - LANES), LANES)
                    idx2_v[sl] = lax.shift_right_logical(idx_v[sl], shift)
                    return c2
                lax.fori_loop(0, n_win, sh, 0)
                pltpu.async_copy(tab_hbm.at[idx2_v], rows_v, sem).wait()
                pltpu.sync_copy(rows_v, out_hbm.at[pl.ds(base, C)])
            return carry

        lax.fori_loop(0, n_it, body, 0)

    return k(table, src, dst)


def _sc_conv1(Y, bidx4, fc4, didx, n_pad):
    E = didx.shape[0]
    per_w = E // NW
    C = 1000
    n_it = per_w // C
    n_win = -(-C // LANES)
    rpw = n_pad // NS
    mesh = plsc.VectorSubcoreMesh(core_axis_name="c", subcore_axis_name="s", num_cores=NC, num_subcores=NS)

    z1 = jnp.zeros((n_pad,), jnp.float32)

    @functools.partial(
        pl.kernel,
        mesh=mesh,
        out_type=jax.ShapeDtypeStruct((NC, n_pad), jnp.float32),
        compiler_params=pltpu.CompilerParams(use_tc_tiling_on_sc=False),
        scratch_types=(
            [pltpu.VMEM((C,), jnp.int32)]
            + [pltpu.VMEM((C,), jnp.float32)] * 4
            + [pltpu.VMEM((C,), jnp.float32)] * 4
            + [
                pltpu.VMEM((C,), jnp.float32),
                pltpu.VMEM_SHARED((n_pad,), jnp.float32),
                pltpu.SemaphoreType.DMA,
            ]
        ),
    )
    def k(y_hbm, b0_h, b1_h, b2_h, b3_h, f0_h, f1_h, f2_h, f3_h, d_h, z_h,
          outP, idx_v, fc0_v, fc1_v, fc2_v, fc3_v, g0_v, g1_v, g2_v, g3_v,
          m_v, acc, sem):
        c = lax.axis_index("c")
        s = lax.axis_index("s")
        wid = s * NC + c
        pltpu.sync_copy(z_h.at[pl.ds(s * rpw, rpw)], acc.at[pl.ds(s * rpw, rpw)])
        plsc.subcore_barrier()
        fcs = (fc0_v, fc1_v, fc2_v, fc3_v)
        gs = (g0_v, g1_v, g2_v, g3_v)

        def chunk(i, carry):
            base = wid * per_w + i * C
            for j, (b_h, f_h) in enumerate(
                    ((b0_h, f0_h), (b1_h, f1_h), (b2_h, f2_h), (b3_h, f3_h))):
                pltpu.sync_copy(b_h.at[pl.ds(base, C)], idx_v)
                pltpu.async_copy(y_hbm.at[idx_v], gs[j], sem).wait()
                pltpu.sync_copy(f_h.at[pl.ds(base, C)], fcs[j])

            def fma(w, c2):
                sl = pl.ds(jnp.minimum(w * LANES, C - LANES), LANES)
                m_v[sl] = (fc0_v[sl] * g0_v[sl] + fc1_v[sl] * g1_v[sl]
                           + fc2_v[sl] * g2_v[sl] + fc3_v[sl] * g3_v[sl])
                return c2
            lax.fori_loop(0, n_win, fma, 0)
            pltpu.sync_copy(d_h.at[pl.ds(base, C)], idx_v)
            pltpu.sync_copy(m_v, acc.at[idx_v], add=True)
            return carry

        lax.fori_loop(0, n_it, chunk, 0)
        plsc.subcore_barrier()
        pltpu.sync_copy(acc.at[pl.ds(s * rpw, rpw)],
                        outP.at[c].at[pl.ds(s * rpw, rpw)])

    b0, b1, b2, b3 = bidx4
    f0, f1, f2, f3 = fc4
    return k(Y, b0, b1, b2, b3, f0, f1, f2, f3, didx, z1)


def _sc_scatter(msg, src, dst, n_pad, level, with_cnt):
    E = msg.shape[0]
    D = msg.shape[1] if msg.ndim == 2 else 1
    per_w = E // NW
    C = 1000
    n_it = per_w // C
    rpw = n_pad // NS
    mesh = plsc.VectorSubcoreMesh(core_axis_name="c", subcore_axis_name="s", num_cores=NC, num_subcores=NS)

    msg_shape = (E, D) if msg.ndim == 2 else (E,)
    acc_shape = (n_pad, D) if msg.ndim == 2 else (n_pad,)
    out_shape = (NC, n_pad, D) if msg.ndim == 2 else (NC, n_pad)
    rows_shape = (C, D) if msg.ndim == 2 else (C,)

    n_win = -(-C // LANES)
    need_tr = level > 0

    out_type = [jax.ShapeDtypeStruct(out_shape, jnp.float32)]
    scratch = [
        pltpu.VMEM((C,), jnp.int32),
        pltpu.VMEM(rows_shape, jnp.float32),
        pltpu.VMEM_SHARED(acc_shape, jnp.float32),
    ]
    if need_tr:
        scratch.append(pltpu.VMEM((C,), jnp.int32))
    if with_cnt:
        out_type.append(jax.ShapeDtypeStruct((NC, n_pad), jnp.float32))
        scratch += [
            pltpu.VMEM((C,), jnp.float32),
            pltpu.VMEM_SHARED((n_pad,), jnp.float32),
        ]
        if need_tr:
            scratch += [
                pltpu.VMEM((C,), jnp.int32),
                pltpu.VMEM((C,), jnp.int32),
            ]

    zD = jnp.zeros(acc_shape, jnp.float32)
    z1 = jnp.zeros((n_pad,), jnp.float32)

    def body(*refs):
        (msg_h, src_h, dst_h, zD_h, z1_h), refs = refs[:5], refs[5:]
        aggP = refs[0]
        cntP = refs[1] if with_cnt else None
        refs = refs[2:] if with_cnt else refs[1:]
        (didx_v, rows_v, accD), refs = refs[:3], refs[3:]
        if need_tr:
            didx2_v, refs = refs[0], refs[1:]
        else:
            didx2_v = didx_v
        if with_cnt:
            (ones_v, acc1), refs = refs[:2], refs[2:]
            if need_tr:
                sidx_v, cidx_v = refs
            else:
                cidx_v = didx_v
        c = lax.axis_index("c")
        s = lax.axis_index("s")
        wid = s * NC + c
        pltpu.sync_copy(zD_h.at[pl.ds(s * rpw, rpw)], accD.at[pl.ds(s * rpw, rpw)])
        if with_cnt:
            pltpu.sync_copy(z1_h.at[pl.ds(s * rpw, rpw)], acc1.at[pl.ds(s * rpw, rpw)])

            def fill_ones(j, c2):
                sl = pl.ds(jnp.minimum(j * LANES, C - LANES), LANES)
                ones_v[sl] = jnp.full((LANES,), 1.0, jnp.float32)
                return c2
            lax.fori_loop(0, n_win, fill_ones, 0)
        plsc.subcore_barrier()

        def chunk(i, carry):
            base = wid * per_w + i * C
            pltpu.sync_copy(dst_h.at[pl.ds(base, C)], didx_v)
            if with_cnt and need_tr:
                pltpu.sync_copy(src_h.at[pl.ds(base, C)], sidx_v)

            if need_tr:
                def tr(j, c2):
                    sl = pl.ds(jnp.minimum(j * LANES, C - LANES), LANES)
                    dv = didx_v[sl]
                    d = lax.shift_right_logical(dv, level)
                    didx2_v[sl] = d
                    if with_cnt:
                        sv = sidx_v[sl]
                        if level == 1:
                            valid = lax.shift_right_logical(sv, 1) != d
                            cidx_v[sl] = jnp.where(valid, d, n_pad - 1)
                        else:
                            v1 = (lax.shift_right_logical(sv, 1)
                                  != lax.shift_right_logical(dv, 1))
                            v2 = lax.shift_right_logical(sv, 2) != d
                            cidx_v[sl] = jnp.where(v1 & v2, d, n_pad - 1)
                    return c2

                lax.fori_loop(0, n_win, tr, 0)
            pltpu.sync_copy(msg_h.at[pl.ds(base, C)], rows_v)
            pltpu.sync_copy(rows_v, accD.at[didx2_v], add=True)
            if with_cnt:
                pltpu.sync_copy(ones_v, acc1.at[cidx_v], add=True)
            return carry

        lax.fori_loop(0, n_it, chunk, 0)
        plsc.subcore_barrier()
        pltpu.sync_copy(accD.at[pl.ds(s * rpw, rpw)],
                        aggP.at[c].at[pl.ds(s * rpw, rpw)])
        if with_cnt:
            pltpu.sync_copy(acc1.at[pl.ds(s * rpw, rpw)],
                            cntP.at[c].at[pl.ds(s * rpw, rpw)])

    k = pl.kernel(body, out_type=tuple(out_type), mesh=mesh,
                  compiler_params=pltpu.CompilerParams(use_tc_tiling_on_sc=False),
                  scratch_types=scratch)
    return k(msg, src, dst, zD, z1)


def _tc_conv_edge(xs, pse, src2d, dst2d, mx, Wf, level, K, b_const):
    E, ci = xs.shape
    co = Wf.shape[1]
    Ksq = K * K
    Eb = 640
    grid = E // Eb

    def body(xs_ref, pse_ref, src_ref, dst_ref, mx_ref, w_ref, out_ref):
        ps = pse_ref[...] / (2.0 * mx_ref[0, 0]) + b_const
        u = jnp.clip(ps, 0.0, 1.0) * (K - 1)
        i0 = jnp.clip(jnp.floor(u), 0.0, float(K - 2))
        frac = u - i0
        i00 = i0[:, 0:1]
        i01 = i0[:, 1:2]
        f0 = frac[:, 0:1]
        f1 = frac[:, 1:2]
        iota_k = lax.broadcasted_iota(jnp.int32, (Eb, Ksq), 1).astype(jnp.float32)
        coeffs = jnp.zeros((Eb, Ksq), jnp.float32)
        for b0 in (0, 1):
            for b1 in (0, 1):
                cidx = (i00 + b0) * K + (i01 + b1)
                fc = (f0 if b0 else 1.0 - f0) * (f1 if b1 else 1.0 - f1)
                coeffs = coeffs + jnp.where(iota_k == cidx, fc, 0.0)
        xsv = xs_ref[...]
        pieces = [coeffs[:, kk:kk + 1] * xsv for kk in range(Ksq)]
        xk = jnp.concatenate(pieces, axis=1)
        msg = jnp.dot(xk, w_ref[...], preferred_element_type=jnp.float32)
        if level == 1:
            sv = src_ref[...]
            dv = dst_ref[...]
            val = (sv // 2 != dv // 2).astype(jnp.float32)
            msg = msg * val
        elif level == 2:
            sv = src_ref[...]
            dv = dst_ref[...]
            val = ((sv // 2 != dv // 2) & (sv // 4 != dv // 4)).astype(jnp.float32)
            msg = msg * val
        out_ref[...] = msg

    return pl.pallas_call(
        body,
        grid=(grid,),
        in_specs=[
            pl.BlockSpec((Eb, ci), lambda i: (i, 0)),
            pl.BlockSpec((Eb, 2), lambda i: (i, 0)),
            pl.BlockSpec((Eb, 1), lambda i: (i, 0)),
            pl.BlockSpec((Eb, 1), lambda i: (i, 0)),
            pl.BlockSpec((1, 1), lambda i: (0, 0), memory_space=pltpu.SMEM),
            pl.BlockSpec((Ksq * ci, co), lambda i: (0, 0)),
        ],
        out_specs=pl.BlockSpec((Eb, co), lambda i: (i, 0)),
        out_shape=jax.ShapeDtypeStruct((E, co), jnp.float32),
    )(xs, pse, src2d, dst2d, mx, Wf)


def _tc_fc(pse, src2d, dst2d, mx, level, K):
    E = pse.shape[0]
    Ksq = K * K
    Eb = 2000
    grid = E // Eb

    def body(pse_ref, src_ref, dst_ref, mx_ref,
             f0_ref, f1_ref, f2_ref, f3_ref,
             b0_ref, b1_ref, b2_ref, b3_ref, d_ref):
        ps = pse_ref[...] / (2.0 * mx_ref[0, 0]) + 0.5
        u = jnp.clip(ps, 0.0, 1.0) * (K - 1)
        i0 = jnp.clip(jnp.floor(u), 0.0, float(K - 2))
        frac = u - i0
        i00 = i0[:, 0:1].astype(jnp.int32)
        i01 = i0[:, 1:2].astype(jnp.int32)
        f0 = frac[:, 0:1]
        f1 = frac[:, 1:2]
        sv = src_ref[...]
        dv = dst_ref[...]
        if level == 1:
            val = (sv // 2 != dv // 2).astype(jnp.float32)
        else:
            val = ((sv // 2 != dv // 2) & (sv // 4 != dv // 4)).astype(jnp.float32)
        sbase = (sv // (1 << level)) * Ksq + i00 * K + i01
        f0_ref[...] = (1.0 - f0) * (1.0 - f1) * val
        f1_ref[...] = (1.0 - f0) * f1 * val
        f2_ref[...] = f0 * (1.0 - f1) * val
        f3_ref[...] = f0 * f1 * val
        b0_ref[...] = sbase
        b1_ref[...] = sbase + 1
        b2_ref[...] = sbase + K
        b3_ref[...] = sbase + K + 1
        d_ref[...] = dv // (1 << level)

    fshape = jax.ShapeDtypeStruct((E, 1), jnp.float32)
    ishape = jax.ShapeDtypeStruct((E, 1), jnp.int32)
    espec = pl.BlockSpec((Eb, 1), lambda i: (i, 0))
    outs = pl.pallas_call(
        body,
        grid=(grid,),
        in_specs=[
            pl.BlockSpec((Eb, 2), lambda i: (i, 0)),
            espec,
            espec,
            pl.BlockSpec((1, 1), lambda i: (0, 0), memory_space=pltpu.SMEM),
        ],
        out_specs=(espec,) * 9,
        out_shape=(fshape,) * 4 + (ishape,) * 5,
    )(pse, src2d, dst2d, mx)
    fc4 = tuple(o.reshape(E) for o in outs[:4])
    bidx4 = tuple(o.reshape(E) for o in outs[4:8])
    didx = outs[8].reshape(E)
    return fc4, bidx4, didx


def _tc_conv_post(aggP, cntP, x, root, bias, Wt=None):
    n, ci = x.shape
    co = root.shape[1]

    def body(*refs):
        if Wt is None:
            agg_ref, cnt_ref, x_ref, r_ref, b_ref, out_ref = refs
        else:
            agg_ref, cnt_ref, x_ref, r_ref, b_ref, wt_ref, out_ref, y_ref = refs
        agg = agg_ref[0] + agg_ref[1]
        cnt = cnt_ref[0] + cnt_ref[1]
        agg = agg / jnp.maximum(cnt, 1.0)
        if ci == 1:
            rootmm = x_ref[...] * r_ref[...]
        else:
            rootmm = jnp.dot(x_ref[...], r_ref[...],
                             preferred_element_type=jnp.float32)
        h = agg + rootmm + b_ref[...]
        h = jnp.where(h > 0, h, jnp.exp(h) - 1.0)
        out_ref[...] = h
        if Wt is not None:
            y_ref[...] = jnp.dot(h, wt_ref[...],
                                 preferred_element_type=jnp.float32)

    out_shape = jax.ShapeDtypeStruct((n, co), jnp.float32)
    if Wt is None:
        return pl.pallas_call(body, out_shape=out_shape)(
            aggP, cntP, x, root, bias.reshape(1, co))
    return pl.pallas_call(
        body,
        out_shape=(out_shape,
                   jax.ShapeDtypeStruct((n, Wt.shape[1]), jnp.float32)),
    )(aggP, cntP, x, root, bias.reshape(1, co), Wt)


def _tc_pools(hr, pr, Wt=None):
    n2, c2 = hr.shape
    co = c2 // 2

    def body(*refs):
        if Wt is None:
            h_ref, p_ref, x_ref, q_ref = refs
        else:
            h_ref, p_ref, wt_ref, x_ref, q_ref, y_ref = refs
        h = h_ref[...]
        x = jnp.maximum(h[:, :co], h[:, co:])
        x_ref[...] = x
        p = p_ref[...]
        q_ref[...] = (p[:, :2] + p[:, 2:]) * 0.5
        if Wt is not None:
            y_ref[...] = jnp.dot(x, wt_ref[...],
                                 preferred_element_type=jnp.float32)

    outs = [jax.ShapeDtypeStruct((n2, co), jnp.float32),
            jax.ShapeDtypeStruct((n2, 2), jnp.float32)]
    if Wt is None:
        return pl.pallas_call(body, out_shape=tuple(outs))(hr, pr)
    outs.append(jax.ShapeDtypeStruct((n2, Wt.shape[1]), jnp.float32))
    return pl.pallas_call(body, out_shape=tuple(outs))(hr, pr, Wt)


def _tc_cart(ps, pd):
    E = ps.shape[0]
    Eb = 2000
    grid = E // Eb

    def body(ps_ref, pd_ref, cart_ref, mx_ref):
        cart = ps_ref[...] - pd_ref[...]
        cart_ref[...] = cart
        m = jnp.max(jnp.abs(cart))

        @pl.when(pl.program_id(0) == 0)
        def _init():
            mx_ref[0, 0] = m

        @pl.when(pl.program_id(0) > 0)
        def _acc():
            mx_ref[0, 0] = jnp.maximum(mx_ref[0, 0], m)

    return pl.pallas_call(
        body,
        grid=(grid,),
        in_specs=[
            pl.BlockSpec((Eb, 2), lambda i: (i, 0)),
            pl.BlockSpec((Eb, 2), lambda i: (i, 0)),
        ],
        out_specs=(
            pl.BlockSpec((Eb, 2), lambda i: (i, 0)),
            pl.BlockSpec((1, 1), lambda i: (0, 0), memory_space=pltpu.SMEM),
        ),
        out_shape=(
            jax.ShapeDtypeStruct((E, 2), jnp.float32),
            jax.ShapeDtypeStruct((1, 1), jnp.float32),
        ),
    )(ps, pd)


def _tc_final(h2, ps2):
    n2 = h2.shape[0]

    def body(h_ref, p_ref, out_ref):
        h = h_ref[...]
        p0 = p_ref[:, 0:1]
        p1 = p_ref[:, 1:2]
        o = jnp.concatenate([h + p0, h + p0, h + p1, h + p1], axis=1)
        out_ref[...] = 1.0 / (1.0 + jnp.exp(-o))

    return pl.pallas_call(
        body, out_shape=jax.ShapeDtypeStruct((n2, 4), jnp.float32))(h2, ps2)


def kernel(x, edge_index, pos, edge_attr,
           W1a, R1a, B1a, W1b, R1b, B1b, W2a, R2a, B2a, W2b, R2b, B2b,
           W3a, R3a, B3a, W3b, R3b, B3b, Wfr, Rfr, Bfr, Wp1, Rp1, Bp1):
    n0 = x.shape[0]
    E = edge_index.shape[1]
    src = edge_index[0].astype(jnp.int32)
    dst = edge_index[1].astype(jnp.int32)
    src2d = src.reshape(E, 1)
    dst2d = dst.reshape(E, 1)
    mx0 = jnp.full((1, 1), 0.5, jnp.float32)

    n1 = n0 // 2
    n2 = n1 // 2
    n0p = _round_up(n0, 128)
    n1p = _round_up(n1, 128)
    n2p = _round_up(n2, 128)

    xs = _sc_gather(x, src, 0)
    msg = _tc_conv_edge(xs, edge_attr, src2d, dst2d, mx0,
                        W1a.reshape(25 * 1, 32), 0, 5, 0.0)
    aggP, cnt0P = _sc_scatter(msg, src, dst, n0p, 0, True)
    cnt0 = cnt0P[:, :n0].reshape(NC, n0, 1)
    h = _tc_conv_post(aggP[:, :n0], cnt0, x, R1a, B1a)

    xs = _sc_gather(h, src, 0)
    msg = _tc_conv_edge(xs, edge_attr, src2d, dst2d, mx0,
                        W1b.reshape(25 * 32, 32), 0, 5, 0.0)
    aggP, = _sc_scatter(msg, src, dst, n0p, 0, False)
    h = _tc_conv_post(aggP[:, :n0], cnt0, h, R1b, B1b)

    x1, pos1, Yp1 = _tc_pools(h.reshape(n1, 64), pos.reshape(n1, 4),
                              Wp1.reshape(9, 32).T)

    ps_g, pd_g = _sc_gather_pair(pos1, src, dst, 1)
    cart1, mx1 = _tc_cart(ps_g, pd_g)
    fc4_1, bidx4_1, didx1 = _tc_fc(cart1, src2d, dst2d, mx1, 1, 3)

    xs = _sc_gather(x1, src, 1)
    msg = _tc_conv_edge(xs, cart1, src2d, dst2d, mx1,
                        W2a.reshape(9 * 32, 64), 1, 3, 0.5)
    aggP, cnt1P = _sc_scatter(msg, src, dst, n1p, 1, True)
    cnt1 = cnt1P[:, :n1].reshape(NC, n1, 1)
    h1 = _tc_conv_post(aggP[:, :n1], cnt1, x1, R2a, B2a)

    xs = _sc_gather(h1, src, 1)
    msg = _tc_conv_edge(xs, cart1, src2d, dst2d, mx1,
                        W2b.reshape(9 * 64, 64), 1, 3, 0.5)
    aggP, = _sc_scatter(msg, src, dst, n1p, 1, False)
    h1 = _tc_conv_post(aggP[:, :n1], cnt1, h1, R2b, B2b)

    aggP1 = _sc_conv1(Yp1.reshape(n1 * 9), bidx4_1, fc4_1, didx1, n1p)
    hp = _tc_conv_post(aggP1[:, :n1].reshape(NC, n1, 1), cnt1, x1, Rp1, Bp1)

    x2, pos2 = _tc_pools(h1.reshape(n2, 128), pos1.reshape(n2, 4))

    ps_g, pd_g = _sc_gather_pair(pos2, src, dst, 2)
    cart2, mx2 = _tc_cart(ps_g, pd_g)
    fc4_2, bidx4_2, didx2 = _tc_fc(cart2, src2d, dst2d, mx2, 2, 3)

    xs = _sc_gather(x2, src, 2)
    msg = _tc_conv_edge(xs, cart2, src2d, dst2d, mx2,
                        W3a.reshape(9 * 64, 64), 2, 3, 0.5)
    aggP, cnt2P = _sc_scatter(msg, src, dst, n2p, 2, True)
    cnt2 = cnt2P[:, :n2].reshape(NC, n2, 1)
    h2 = _tc_conv_post(aggP[:, :n2], cnt2, x2, R3a, B3a)

    xs = _sc_gather(h2, src, 2)
    msg = _tc_conv_edge(xs, cart2, src2d, dst2d, mx2,
                        W3b.reshape(9 * 64, 64), 2, 3, 0.5)
    aggP, = _sc_scatter(msg, src, dst, n2p, 2, False)
    h2, Yfr = _tc_conv_post(aggP[:, :n2], cnt2, h2, R3b, B3b,
                            Wfr.reshape(9, 64).T)

    aggF = _sc_conv1(Yfr.reshape(n2 * 9), bidx4_2, fc4_2, didx2, n2p)
    hfr = _tc_conv_post(aggF[:, :n2].reshape(NC, n2, 1), cnt2, h2, Rfr, Bfr)

    out4 = _tc_final(hfr, hp.reshape(n2, 2))
    return out4.reshape(n0, 1)
